# baseline (speedup 1.0000x reference)
import functools
import jax
import jax.numpy as jnp
from jax import lax
from jax.experimental import pallas as pl
from jax.experimental.pallas import tpu as pltpu
from jax.experimental.pallas import tpu_sc as plsc


def _bcast_kernel(tab_ref, out_ref):
    out_ref[...] = jnp.broadcast_to(tab_ref[...][None, :, :], out_ref.shape)


def _tc_part(table, rows, seqs_len, num_units):
    BB = 128
    return pl.pallas_call(
        _bcast_kernel,
        grid=(rows // BB,),
        in_specs=[pl.BlockSpec((seqs_len, num_units), lambda i: (0, 0))],
        out_specs=pl.BlockSpec((BB, seqs_len, num_units), lambda i: (i, 0, 0)),
        out_shape=jax.ShapeDtypeStruct((rows, seqs_len, num_units), table.dtype),
    )(table)


def _sc_part(table, rows, seqs_len, num_units):
    NC, NS = 2, 16
    NW = NC * NS
    b_per_w = rows // NW
    mesh = plsc.VectorSubcoreMesh(core_axis_name="c", subcore_axis_name="s")

    @functools.partial(
        pl.kernel,
        mesh=mesh,
        out_type=jax.ShapeDtypeStruct((rows, seqs_len, num_units), jnp.float32),
        scratch_types=[pltpu.VMEM((seqs_len, num_units), jnp.float32)],
    )
    def k(table_hbm, out_hbm, tab_v):
        wid = lax.axis_index("s") * NC + lax.axis_index("c")
        base = wid * b_per_w
        pltpu.sync_copy(table_hbm, tab_v)

        def body(i, carry):
            pltpu.sync_copy(tab_v, out_hbm.at[base + i])
            return carry

        lax.fori_loop(0, b_per_w, body, 0)

    return k(table)


def kernel(inputs, pembs_weight):
    batch_size, seqs_len = inputs.shape[:2]
    num_units = pembs_weight.shape[1]
    table = pembs_weight[:seqs_len]

    K = 1792
    sc = _sc_part(table, K, seqs_len, num_units)
    tc = _tc_part(table, batch_size - K, seqs_len, num_units)
    return jnp.concatenate([sc, tc], axis=0)


# speedup vs baseline: 2.6924x; 2.6924x over previous
import functools
import jax
import jax.numpy as jnp
from jax import lax
from jax.experimental import pallas as pl
from jax.experimental.pallas import tpu as pltpu
from jax.experimental.pallas import tpu_sc as plsc


def kernel(inputs, pembs_weight):
    batch_size, seqs_len = inputs.shape[:2]
    num_units = pembs_weight.shape[1]

    NC, NS = 2, 16
    NW = NC * NS
    b_per_w = batch_size // NW

    mesh = plsc.VectorSubcoreMesh(core_axis_name="c", subcore_axis_name="s")

    @functools.partial(
        pl.kernel,
        mesh=mesh,
        out_type=jax.ShapeDtypeStruct((batch_size, seqs_len, num_units), jnp.float32),
        scratch_types=[pltpu.VMEM((seqs_len, num_units), jnp.float32)],
    )
    def k(table_hbm, out_hbm, tab_v):
        wid = lax.axis_index("s") * NC + lax.axis_index("c")
        base = wid * b_per_w
        pltpu.sync_copy(table_hbm.at[pl.ds(0, seqs_len)], tab_v)

        def body(i, carry):
            pltpu.sync_copy(tab_v, out_hbm.at[base + i])
            return carry

        lax.fori_loop(0, b_per_w, body, 0)

    return k(pembs_weight)
